# baseline (speedup 1.0000x reference)
import functools
import math

import jax
import jax.numpy as jnp
from jax.experimental import pallas as pl

B = 128
S = 32
D_IN = 512
HIDDEN = 512
D_K = 512
D_V = 512
NUM_K = 16
ACTIVE = 8

BB = 16


def _rim_kernel(x_ref, h_ref, wq_ref, wk_ref, wv_ref, wx_ref, wh_ref,
                bx_ref, bh_ref, out_ref):
    x = x_ref[...].reshape(BB * S, D_IN)
    h = h_ref[...].reshape(BB * NUM_K, HIDDEN)

    k = jnp.dot(x, wk_ref[...], preferred_element_type=jnp.float32)
    v = jnp.dot(x.astype(jnp.bfloat16), wv_ref[...],
                preferred_element_type=jnp.float32)
    q = jnp.dot(h, wq_ref[...], preferred_element_type=jnp.float32)

    sim = jnp.dot(q, k.T, preferred_element_type=jnp.float32) * (
        1.0 / math.sqrt(D_K))
    row_b = jax.lax.broadcasted_iota(jnp.int32, sim.shape, 0) // NUM_K
    col_b = jax.lax.broadcasted_iota(jnp.int32, sim.shape, 1) // S
    sim = jnp.where(row_b == col_b, sim, -1e30)

    m = jnp.maximum(jnp.max(sim, axis=1, keepdims=True), 0.0)
    e = jnp.exp(sim - m)
    e_null = jnp.exp(-m)
    denom = jnp.sum(e, axis=1, keepdims=True) + e_null
    p = e / denom
    null_attn = (e_null / denom).reshape(BB, NUM_K)

    a = null_attn
    rank = jnp.zeros((BB, NUM_K), dtype=jnp.float32)
    col = jax.lax.broadcasted_iota(jnp.int32, (BB, NUM_K), 1)
    for j in range(NUM_K):
        aj = a[:, j:j + 1]
        cmp = (aj < a) | ((aj == a) & (j < col))
        rank = rank + cmp.astype(jnp.float32)
    mask = (rank < ACTIVE).astype(jnp.float32)
    mask_rows = mask.reshape(BB * NUM_K, 1)

    attended = jnp.dot(p.astype(jnp.bfloat16), v.astype(jnp.bfloat16),
                       preferred_element_type=jnp.float32)
    attended = attended * mask_rows

    gates_x = jnp.dot(attended.astype(jnp.bfloat16), wx_ref[...],
                      preferred_element_type=jnp.float32) + bx_ref[...]
    gates_h = jnp.dot(h.astype(jnp.bfloat16), wh_ref[...],
                      preferred_element_type=jnp.float32) + bh_ref[...]
    xr = gates_x[:, :HIDDEN]
    xz = gates_x[:, HIDDEN:2 * HIDDEN]
    xn = gates_x[:, 2 * HIDDEN:]
    hr = gates_h[:, :HIDDEN]
    hz = gates_h[:, HIDDEN:2 * HIDDEN]
    hn = gates_h[:, 2 * HIDDEN:]
    r = jax.nn.sigmoid(xr + hr)
    z = jax.nn.sigmoid(xz + hz)
    n = jnp.tanh(xn + r * hn)
    new_h = (1.0 - z) * n + z * h
    out = mask_rows * new_h + (1.0 - mask_rows) * h
    out_ref[...] = out.reshape(BB, NUM_K, HIDDEN)


@functools.partial(jax.jit, static_argnames=("interpret",))
def _run(input, init_hidden, W_q, W_k, W_v, W_x, W_h, b_x, b_h,
         interpret=False):
    grid = (B // BB,)
    out = pl.pallas_call(
        _rim_kernel,
        grid=grid,
        in_specs=[
            pl.BlockSpec((BB, S, D_IN), lambda i: (i, 0, 0)),
            pl.BlockSpec((BB, NUM_K, HIDDEN), lambda i: (i, 0, 0)),
            pl.BlockSpec((HIDDEN, D_K), lambda i: (0, 0)),
            pl.BlockSpec((D_IN, D_K), lambda i: (0, 0)),
            pl.BlockSpec((D_IN, D_V), lambda i: (0, 0)),
            pl.BlockSpec((D_V, 3 * HIDDEN), lambda i: (0, 0)),
            pl.BlockSpec((HIDDEN, 3 * HIDDEN), lambda i: (0, 0)),
            pl.BlockSpec((1, 3 * HIDDEN), lambda i: (0, 0)),
            pl.BlockSpec((1, 3 * HIDDEN), lambda i: (0, 0)),
        ],
        out_specs=pl.BlockSpec((BB, NUM_K, HIDDEN), lambda i: (i, 0, 0)),
        out_shape=jax.ShapeDtypeStruct((B, NUM_K, HIDDEN), jnp.float32),
        interpret=interpret,
    )(input, init_hidden, W_q, W_k,
      W_v.astype(jnp.bfloat16), W_x.astype(jnp.bfloat16),
      W_h.astype(jnp.bfloat16),
      b_x.reshape(1, 3 * HIDDEN), b_h.reshape(1, 3 * HIDDEN))
    return out


def kernel(input, init_hidden, W_q, W_k, W_v, W_x, W_h, b_x, b_h):
    return _run(input, init_hidden, W_q, W_k, W_v, W_x, W_h, b_x, b_h)


# speedup vs baseline: 1.1865x; 1.1865x over previous
import functools
import math

import jax
import jax.numpy as jnp
from jax.experimental import pallas as pl

B = 128
S = 32
D_IN = 512
HIDDEN = 512
D_K = 512
D_V = 512
NUM_K = 16
ACTIVE = 8

BB = 16


def _rim_kernel(x_ref, h_ref, wq_ref, wk_ref, wv_ref, wx_ref, wh_ref,
                bx_ref, bh_ref, out_ref):
    x = x_ref[...].reshape(BB * S, D_IN)
    h = h_ref[...].reshape(BB * NUM_K, HIDDEN)

    k = jnp.dot(x, wk_ref[...], preferred_element_type=jnp.float32)
    v = jnp.dot(x, wv_ref[...], preferred_element_type=jnp.float32)
    q = jnp.dot(h, wq_ref[...], preferred_element_type=jnp.float32)

    sim = jnp.dot(q, k.T, preferred_element_type=jnp.float32) * (
        1.0 / math.sqrt(D_K))
    row_b = jax.lax.broadcasted_iota(jnp.int32, sim.shape, 0) // NUM_K
    col_b = jax.lax.broadcasted_iota(jnp.int32, sim.shape, 1) // S
    sim = jnp.where(row_b == col_b, sim, -1e30)

    m = jnp.maximum(jnp.max(sim, axis=1, keepdims=True), 0.0)
    e = jnp.exp(sim - m)
    e_null = jnp.exp(-m)
    denom = jnp.sum(e, axis=1, keepdims=True) + e_null
    p = e / denom
    null_attn = (e_null / denom).reshape(BB, NUM_K)

    a = null_attn
    rank = jnp.zeros((BB, NUM_K), dtype=jnp.float32)
    col = jax.lax.broadcasted_iota(jnp.int32, (BB, NUM_K), 1)
    for j in range(NUM_K):
        aj = a[:, j:j + 1]
        cmp = (aj < a) | ((aj == a) & (j < col))
        rank = rank + cmp.astype(jnp.float32)
    mask = (rank < ACTIVE).astype(jnp.float32)
    mask_rows = mask.reshape(BB * NUM_K, 1)

    attended = jnp.dot(p, v, preferred_element_type=jnp.float32)
    attended = attended * mask_rows

    gates_x = jnp.dot(attended, wx_ref[...],
                      preferred_element_type=jnp.float32) + bx_ref[...]
    gates_h = jnp.dot(h, wh_ref[...],
                      preferred_element_type=jnp.float32) + bh_ref[...]
    xr = gates_x[:, :HIDDEN]
    xz = gates_x[:, HIDDEN:2 * HIDDEN]
    xn = gates_x[:, 2 * HIDDEN:]
    hr = gates_h[:, :HIDDEN]
    hz = gates_h[:, HIDDEN:2 * HIDDEN]
    hn = gates_h[:, 2 * HIDDEN:]
    r = jax.nn.sigmoid(xr + hr)
    z = jax.nn.sigmoid(xz + hz)
    n = jnp.tanh(xn + r * hn)
    new_h = (1.0 - z) * n + z * h
    out = mask_rows * new_h + (1.0 - mask_rows) * h
    out_ref[...] = out.reshape(BB, NUM_K, HIDDEN)


@functools.partial(jax.jit, static_argnames=("interpret",))
def _run(input, init_hidden, W_q, W_k, W_v, W_x, W_h, b_x, b_h,
         interpret=False):
    grid = (B // BB,)
    out = pl.pallas_call(
        _rim_kernel,
        grid=grid,
        in_specs=[
            pl.BlockSpec((BB, S, D_IN), lambda i: (i, 0, 0)),
            pl.BlockSpec((BB, NUM_K, HIDDEN), lambda i: (i, 0, 0)),
            pl.BlockSpec((HIDDEN, D_K), lambda i: (0, 0)),
            pl.BlockSpec((D_IN, D_K), lambda i: (0, 0)),
            pl.BlockSpec((D_IN, D_V), lambda i: (0, 0)),
            pl.BlockSpec((D_V, 3 * HIDDEN), lambda i: (0, 0)),
            pl.BlockSpec((HIDDEN, 3 * HIDDEN), lambda i: (0, 0)),
            pl.BlockSpec((1, 3 * HIDDEN), lambda i: (0, 0)),
            pl.BlockSpec((1, 3 * HIDDEN), lambda i: (0, 0)),
        ],
        out_specs=pl.BlockSpec((BB, NUM_K, HIDDEN), lambda i: (i, 0, 0)),
        out_shape=jax.ShapeDtypeStruct((B, NUM_K, HIDDEN), jnp.float32),
        interpret=interpret,
    )(input, init_hidden, W_q, W_k, W_v, W_x, W_h,
      b_x.reshape(1, 3 * HIDDEN), b_h.reshape(1, 3 * HIDDEN))
    return out


def kernel(input, init_hidden, W_q, W_k, W_v, W_x, W_h, b_x, b_h):
    return _run(input, init_hidden, W_q, W_k, W_v, W_x, W_h, b_x, b_h)


# speedup vs baseline: 1.3291x; 1.1202x over previous
import functools
import math

import jax
import jax.numpy as jnp
from jax.experimental import pallas as pl

B = 128
S = 32
D_IN = 512
HIDDEN = 512
D_K = 512
D_V = 512
NUM_K = 16
ACTIVE = 8

BB = 32
F32 = jnp.float32


def _rim_kernel(x_ref, h_ref, wq_ref, wk_ref, wv_ref, wx_ref, wh_ref,
                bx_ref, bh_ref, out_ref):
    rows = BB * NUM_K
    half = rows // 2
    x = x_ref[...].reshape(BB * S, D_IN)
    h = h_ref[...].reshape(rows, HIDDEN)

    k = jnp.dot(x, wk_ref[...], preferred_element_type=F32)
    v = jnp.dot(x, wv_ref[...], preferred_element_type=F32)
    q = jnp.dot(h, wq_ref[...], preferred_element_type=F32)

    sim = jnp.dot(q, k.T, preferred_element_type=F32) * (
        1.0 / math.sqrt(D_K))
    row_b = jax.lax.broadcasted_iota(jnp.int32, sim.shape, 0) // NUM_K
    col_b = jax.lax.broadcasted_iota(jnp.int32, sim.shape, 1) // S
    sim = jnp.where(row_b == col_b, sim, -1e30)

    m = jnp.maximum(jnp.max(sim, axis=1, keepdims=True), 0.0)
    e = jnp.exp(sim - m)
    e_null = jnp.exp(-m)
    denom = jnp.sum(e, axis=1, keepdims=True) + e_null
    p = e / denom
    a = (e_null / denom).reshape(BB, NUM_K)

    tk = a
    for _ in range(4):
        tk = jnp.concatenate([tk, tk], axis=1)
    tj = jnp.broadcast_to(a[:, :, None],
                          (BB, NUM_K, NUM_K)).reshape(BB, NUM_K * NUM_K)
    lane = jax.lax.broadcasted_iota(jnp.int32, (BB, NUM_K * NUM_K), 1)
    jlt = (lane // NUM_K) < (lane % NUM_K)
    cmp = ((tj < tk) | ((tj == tk) & jlt)).astype(F32)
    r_i = jax.lax.broadcasted_iota(jnp.int32, (NUM_K * NUM_K, NUM_K), 0)
    c_i = jax.lax.broadcasted_iota(jnp.int32, (NUM_K * NUM_K, NUM_K), 1)
    msum = (r_i % NUM_K == c_i).astype(F32)
    rank = jnp.dot(cmp, msum, preferred_element_type=F32)

    base = jax.lax.broadcasted_iota(
        jnp.int32, (BB, NUM_K), 0).astype(F32) * ACTIVE
    cidm = jnp.where(rank < ACTIVE, base + rank, -1.0)

    g_row = jax.lax.broadcasted_iota(jnp.int32, (rows, BB), 0)
    g_col = jax.lax.broadcasted_iota(jnp.int32, (rows, BB), 1)
    g = (g_row // NUM_K == g_col).astype(F32)
    r_row = jax.lax.broadcasted_iota(jnp.int32, (rows, NUM_K), 0)
    r_col = jax.lax.broadcasted_iota(jnp.int32, (rows, NUM_K), 1)
    oh = (r_row % NUM_K == r_col).astype(F32)
    y = jnp.dot(g, cidm, preferred_element_type=F32)
    cid_rows = jnp.sum(y * oh, axis=1, keepdims=True)
    col_i = jax.lax.broadcasted_iota(
        jnp.int32, (rows, half), 1).astype(F32)
    ct = (cid_rows == col_i).astype(F32)

    p_c = jax.lax.dot_general(ct, p, (((0,), (0,)), ((), ())),
                              preferred_element_type=F32)
    h_c = jax.lax.dot_general(ct, h, (((0,), (0,)), ((), ())),
                              preferred_element_type=F32)
    attended_c = jnp.dot(p_c, v, preferred_element_type=F32)

    gates_x = jnp.dot(attended_c, wx_ref[...],
                      preferred_element_type=F32) + bx_ref[...]
    gates_h = jnp.dot(h_c, wh_ref[...],
                      preferred_element_type=F32) + bh_ref[...]
    xr = gates_x[:, :HIDDEN]
    xz = gates_x[:, HIDDEN:2 * HIDDEN]
    xn = gates_x[:, 2 * HIDDEN:]
    hr = gates_h[:, :HIDDEN]
    hz = gates_h[:, HIDDEN:2 * HIDDEN]
    hn = gates_h[:, 2 * HIDDEN:]
    r = jax.nn.sigmoid(xr + hr)
    z = jax.nn.sigmoid(xz + hz)
    n = jnp.tanh(xn + r * hn)
    delta = (1.0 - z) * (n - h_c)
    out = h + jnp.dot(ct, delta, preferred_element_type=F32)
    out_ref[...] = out.reshape(BB, NUM_K, HIDDEN)


@functools.partial(jax.jit, static_argnames=("interpret",))
def _run(input, init_hidden, W_q, W_k, W_v, W_x, W_h, b_x, b_h,
         interpret=False):
    grid = (B // BB,)
    out = pl.pallas_call(
        _rim_kernel,
        grid=grid,
        in_specs=[
            pl.BlockSpec((BB, S, D_IN), lambda i: (i, 0, 0)),
            pl.BlockSpec((BB, NUM_K, HIDDEN), lambda i: (i, 0, 0)),
            pl.BlockSpec((HIDDEN, D_K), lambda i: (0, 0)),
            pl.BlockSpec((D_IN, D_K), lambda i: (0, 0)),
            pl.BlockSpec((D_IN, D_V), lambda i: (0, 0)),
            pl.BlockSpec((D_V, 3 * HIDDEN), lambda i: (0, 0)),
            pl.BlockSpec((HIDDEN, 3 * HIDDEN), lambda i: (0, 0)),
            pl.BlockSpec((1, 3 * HIDDEN), lambda i: (0, 0)),
            pl.BlockSpec((1, 3 * HIDDEN), lambda i: (0, 0)),
        ],
        out_specs=pl.BlockSpec((BB, NUM_K, HIDDEN), lambda i: (i, 0, 0)),
        out_shape=jax.ShapeDtypeStruct((B, NUM_K, HIDDEN), jnp.float32),
        interpret=interpret,
    )(input, init_hidden, W_q, W_k, W_v, W_x, W_h,
      b_x.reshape(1, 3 * HIDDEN), b_h.reshape(1, 3 * HIDDEN))
    return out


def kernel(input, init_hidden, W_q, W_k, W_v, W_x, W_h, b_x, b_h):
    return _run(input, init_hidden, W_q, W_k, W_v, W_x, W_h, b_x, b_h)


# speedup vs baseline: 1.6554x; 1.2454x over previous
import functools
import math

import jax
import jax.numpy as jnp
from jax.experimental import pallas as pl
from jax.experimental.pallas import tpu as pltpu

B = 128
S = 32
D_IN = 512
HIDDEN = 512
D_K = 512
D_V = 512
NUM_K = 16
ACTIVE = 8

BB = 32
GA = 16
F32 = jnp.float32


def _rim_kernel(x_ref, h_ref, wq_ref, wk_ref, wv_ref, wx_ref, wh_ref,
                bx_ref, bh_ref, out_ref, wx_vmem, wh_vmem, sem_x, sem_h):
    rows = BB * NUM_K
    half = rows // 2

    cp_x = pltpu.make_async_copy(wx_ref, wx_vmem, sem_x)
    cp_h = pltpu.make_async_copy(wh_ref, wh_vmem, sem_h)

    @pl.when(pl.program_id(0) == 0)
    def _start():
        cp_x.start()
        cp_h.start()

    x = x_ref[...].reshape(BB * S, D_IN)
    h = h_ref[...].reshape(rows, HIDDEN)

    k = jnp.dot(x, wk_ref[...], preferred_element_type=F32)
    q = jnp.dot(h, wq_ref[...], preferred_element_type=F32)

    grows = GA * NUM_K
    gcols = GA * S
    inv = 1.0 / math.sqrt(D_K)
    row_b = jax.lax.broadcasted_iota(jnp.int32, (grows, gcols), 0) // NUM_K
    col_b = jax.lax.broadcasted_iota(jnp.int32, (grows, gcols), 1) // S
    valid = row_b == col_b
    ps = []
    nulls = []
    for gi in range(BB // GA):
        qg = q[gi * grows:(gi + 1) * grows, :]
        kg = k[gi * gcols:(gi + 1) * gcols, :]
        sim = jnp.dot(qg, kg.T, preferred_element_type=F32) * inv
        sim = jnp.where(valid, sim, -1e30)
        m = jnp.maximum(jnp.max(sim, axis=1, keepdims=True), 0.0)
        e = jnp.exp(sim - m)
        e_null = jnp.exp(-m)
        denom = jnp.sum(e, axis=1, keepdims=True) + e_null
        ps.append(e / denom)
        nulls.append((e_null / denom).reshape(GA, NUM_K))
    a = jnp.concatenate(nulls, axis=0)

    tk = a
    for _ in range(4):
        tk = jnp.concatenate([tk, tk], axis=1)
    tj = jnp.broadcast_to(a[:, :, None],
                          (BB, NUM_K, NUM_K)).reshape(BB, NUM_K * NUM_K)
    lane = jax.lax.broadcasted_iota(jnp.int32, (BB, NUM_K * NUM_K), 1)
    jlt = (lane // NUM_K) < (lane % NUM_K)
    cmp = ((tj < tk) | ((tj == tk) & jlt)).astype(F32)
    r_i = jax.lax.broadcasted_iota(jnp.int32, (NUM_K * NUM_K, NUM_K), 0)
    c_i = jax.lax.broadcasted_iota(jnp.int32, (NUM_K * NUM_K, NUM_K), 1)
    msum = (r_i % NUM_K == c_i).astype(F32)
    rank = jnp.dot(cmp, msum, preferred_element_type=F32)

    cidm = jnp.where(rank < ACTIVE, rank, -1.0)

    g_row = jax.lax.broadcasted_iota(jnp.int32, (rows, BB), 0)
    g_col = jax.lax.broadcasted_iota(jnp.int32, (rows, BB), 1)
    g = (g_row // NUM_K == g_col).astype(F32)
    r_row = jax.lax.broadcasted_iota(jnp.int32, (rows, NUM_K), 0)
    r_col = jax.lax.broadcasted_iota(jnp.int32, (rows, NUM_K), 1)
    oh = (r_row % NUM_K == r_col).astype(F32)
    y = jnp.dot(g, cidm, preferred_element_type=F32)
    rank_rows = jnp.sum(y * oh, axis=1, keepdims=True)
    row_base = (jax.lax.broadcasted_iota(jnp.int32, (rows, 1), 0)
                // NUM_K).astype(F32) * ACTIVE
    cid_rows = jnp.where(rank_rows >= 0.0, row_base + rank_rows, -1.0)
    col_i = jax.lax.broadcasted_iota(
        jnp.int32, (rows, half), 1).astype(F32)
    ct = (cid_rows == col_i).astype(F32)

    ghalf = grows // 2
    pxs = []
    for gi in range(BB // GA):
        ctg = ct[gi * grows:(gi + 1) * grows, gi * ghalf:(gi + 1) * ghalf]
        p_cg = jax.lax.dot_general(ctg, ps[gi], (((0,), (0,)), ((), ())),
                                   preferred_element_type=F32)
        xg = x[gi * gcols:(gi + 1) * gcols, :]
        pxs.append(jnp.dot(p_cg, xg, preferred_element_type=F32))
    px = jnp.concatenate(pxs, axis=0)
    hcs = []
    for gi in range(BB // GA):
        ctg = ct[gi * grows:(gi + 1) * grows, gi * ghalf:(gi + 1) * ghalf]
        hg = h[gi * grows:(gi + 1) * grows, :]
        hcs.append(jax.lax.dot_general(ctg, hg, (((0,), (0,)), ((), ())),
                                       preferred_element_type=F32))
    h_c = jnp.concatenate(hcs, axis=0)
    attended_c = jnp.dot(px, wv_ref[...],
                         preferred_element_type=F32)

    @pl.when(pl.program_id(0) == 0)
    def _wait():
        cp_x.wait()
        cp_h.wait()

    gates_x = jnp.dot(attended_c, wx_vmem[...],
                      preferred_element_type=F32) + bx_ref[...]
    gates_h = jnp.dot(h_c, wh_vmem[...],
                      preferred_element_type=F32) + bh_ref[...]
    xr = gates_x[:, :HIDDEN]
    xz = gates_x[:, HIDDEN:2 * HIDDEN]
    xn = gates_x[:, 2 * HIDDEN:]
    hr = gates_h[:, :HIDDEN]
    hz = gates_h[:, HIDDEN:2 * HIDDEN]
    hn = gates_h[:, 2 * HIDDEN:]
    r = jax.nn.sigmoid(xr + hr)
    z = jax.nn.sigmoid(xz + hz)
    n = jnp.tanh(xn + r * hn)
    delta = (1.0 - z) * (n - h_c)
    outs = []
    for gi in range(BB // GA):
        ctg = ct[gi * grows:(gi + 1) * grows, gi * ghalf:(gi + 1) * ghalf]
        dg = delta[gi * ghalf:(gi + 1) * ghalf, :]
        hg = h[gi * grows:(gi + 1) * grows, :]
        outs.append(hg + jnp.dot(ctg, dg, preferred_element_type=F32))
    out = jnp.concatenate(outs, axis=0)
    out_ref[...] = out.reshape(BB, NUM_K, HIDDEN)


@functools.partial(jax.jit, static_argnames=("interpret",))
def _run(input, init_hidden, W_q, W_k, W_v, W_x, W_h, b_x, b_h,
         interpret=False):
    grid = (B // BB,)
    out = pl.pallas_call(
        _rim_kernel,
        grid=grid,
        in_specs=[
            pl.BlockSpec((BB, S, D_IN), lambda i: (i, 0, 0)),
            pl.BlockSpec((BB, NUM_K, HIDDEN), lambda i: (i, 0, 0)),
            pl.BlockSpec((HIDDEN, D_K), lambda i: (0, 0)),
            pl.BlockSpec((D_IN, D_K), lambda i: (0, 0)),
            pl.BlockSpec((D_IN, D_V), lambda i: (0, 0)),
            pl.BlockSpec(memory_space=pl.ANY),
            pl.BlockSpec(memory_space=pl.ANY),
            pl.BlockSpec((1, 3 * HIDDEN), lambda i: (0, 0)),
            pl.BlockSpec((1, 3 * HIDDEN), lambda i: (0, 0)),
        ],
        out_specs=pl.BlockSpec((BB, NUM_K, HIDDEN), lambda i: (i, 0, 0)),
        out_shape=jax.ShapeDtypeStruct((B, NUM_K, HIDDEN), jnp.float32),
        scratch_shapes=[
            pltpu.VMEM((D_V, 3 * HIDDEN), jnp.float32),
            pltpu.VMEM((HIDDEN, 3 * HIDDEN), jnp.float32),
            pltpu.SemaphoreType.DMA,
            pltpu.SemaphoreType.DMA,
        ],
        interpret=interpret,
    )(input, init_hidden, W_q, W_k, W_v, W_x, W_h,
      b_x.reshape(1, 3 * HIDDEN), b_h.reshape(1, 3 * HIDDEN))
    return out


def kernel(input, init_hidden, W_q, W_k, W_v, W_x, W_h, b_x, b_h):
    return _run(input, init_hidden, W_q, W_k, W_v, W_x, W_h, b_x, b_h)
